# baseline (speedup 1.0000x reference)
import functools

import jax
import jax.numpy as jnp
import numpy as np
from jax import lax
from jax.experimental import pallas as pl
from jax.experimental.pallas import tpu as pltpu
from jax.experimental.pallas import tpu_sc as plsc

N_ROWS = 1_000_000
N_COLS = 4
N_FLAT = N_ROWS * N_COLS
NUM_DIST = 64
NUM_ANGLE = 32

NC, NS, L = 2, 16, 16
NW = NC * NS
B_ROWS = 512
B_FLAT = B_ROWS * N_COLS
N_GATHER = B_FLAT // 128
N_CHUNKS = (N_ROWS + B_ROWS - 1) // B_ROWS
TRIPS = (N_CHUNKS + NW - 1) // NW
LAST_BASE = N_ROWS - B_ROWS

def _body(emb_hbm, tab_hbm, out_hbm, embv, idx1, rows, semg):
    c = lax.axis_index("c")
    s = lax.axis_index("s")
    wid = s * NC + c

    lane = lax.iota(jnp.int32, L)
    is_dist = (lane % 4) == 3
    perm = lane + jnp.where(is_dist, -3, 1)
    scale = jnp.where(is_dist, jnp.float32(64.0), jnp.float32(32.0))
    maxv = jnp.where(is_dist, 63, 31)
    offv = jnp.where(is_dist, 32, 0)

    def chunk_body(k, carry):
        i = wid + k * NW

        @pl.when(i < N_CHUNKS)
        def _do():
            base = jnp.minimum(i * B_ROWS, LAST_BASE)
            fb = base * N_COLS
            pltpu.sync_copy(emb_hbm.at[pl.ds(fb, B_FLAT)], embv)

            def vec_body(v, inner):
                e = plsc.load_gather(embv, [perm + v * L])
                y = e * scale
                t = y.astype(jnp.int32)
                tf = t.astype(jnp.float32)
                idx = jnp.where(y > tf, t, t - 1)
                idx = jnp.minimum(jnp.maximum(idx, 0), maxv) + offv
                idx1[pl.ds(v * L, L)] = idx
                return inner

            lax.fori_loop(0, B_FLAT // L, vec_body, 0)

            copies = [
                pltpu.async_copy(
                    tab_hbm.at[idx1.at[pl.ds(j * 128, 128)]],
                    rows.at[pl.ds(j * 128, 128)],
                    semg,
                )
                for j in range(N_GATHER)
            ]
            for cp in copies:
                cp.wait()

            pltpu.sync_copy(rows, out_hbm.at[pl.ds(fb, B_FLAT)])

        return carry

    lax.fori_loop(0, TRIPS, chunk_body, 0)


@jax.jit
def _sc_call(emb_flat, table):
    mesh = plsc.VectorSubcoreMesh(
        core_axis_name="c", subcore_axis_name="s", num_cores=NC, num_subcores=NS
    )
    return pl.kernel(
        _body,
        out_type=jax.ShapeDtypeStruct((N_FLAT, 16), jnp.float32),
        mesh=mesh,
        compiler_params=pltpu.CompilerParams(
            needs_layout_passes=False, use_tc_tiling_on_sc=False
        ),
        scratch_types=[
            pltpu.VMEM((B_FLAT,), jnp.float32),
            pltpu.VMEM((B_FLAT,), jnp.int32),
            pltpu.VMEM((B_FLAT, 16), jnp.float32),
            pltpu.SemaphoreType.DMA,
        ],
    )(emb_flat, table)


def kernel(embeddings, W_dist, b_dist, W_angle, b_angle):
    table = jnp.concatenate(
        [W_angle + b_angle[None, :], W_dist + b_dist[None, :]], axis=0
    )
    emb_flat = embeddings.reshape(-1)
    out = _sc_call(emb_flat, table)
    return out.reshape(embeddings.shape[0], 64)
